# baseline (speedup 1.0000x reference)
import functools

import jax
import jax.numpy as jnp
from jax import lax
from jax.experimental import pallas as pl
from jax.experimental.pallas import tpu as pltpu
from jax.experimental.pallas import tpu_sc as plsc

B = 16384
N_STIMULI = 30
N_DIM = 10
N_REF = 4
NV = N_STIMULI + 1

NC = 2
NS = 16
L = 16
NW = NC * NS
TPW = B // NW
CHUNKS = TPW // L


def _sim_body(table_ref, w0_ref, w1_ref, out_ref):
    t = table_ref[...]
    z1 = t[:, None, :]
    z2 = t[None, :, :]
    sq = (z1 - z2) * (z1 - z2)
    for g in range(2):
        w = (w0_ref if g == 0 else w1_ref)[...]
        d2 = jnp.sum(sq * w[None, None, :], axis=-1)
        s = jnp.exp(-jnp.sqrt(d2 + 1e-12))
        out_ref[g * 32:g * 32 + NV, :NV] = s


_sim_tables = pl.pallas_call(
    _sim_body,
    out_shape=jax.ShapeDtypeStruct((64, 128), jnp.float32),
)


def _sc_body(sim_hbm, sst_hbm, gate_hbm, out_hbm,
             sim_v, ss_v, gate_v, o_v, sem):
    cid = lax.axis_index("c")
    sid = lax.axis_index("s")
    wid = sid * NC + cid
    base = wid * TPW

    c1 = pltpu.async_copy(sim_hbm, sim_v, sem)
    c2 = pltpu.async_copy(sst_hbm.at[:, pl.ds(base, TPW)], ss_v, sem)
    c3 = pltpu.async_copy(gate_hbm.at[pl.ds(base, TPW)], gate_v, sem)
    c1.wait()
    c2.wait()
    c3.wait()

    @plsc.parallel_loop(0, CHUNKS, unroll=8)
    def chunk(g):
        sl = pl.ds(g * L, L)
        gq = gate_v[sl] * 4096 + ss_v[0, sl] * 128
        s_vals = [plsc.load_gather(sim_v, [gq + ss_v[1 + j, sl]])
                  for j in range(N_REF)]
        tot = (s_vals[0] + s_vals[1]) + (s_vals[2] + s_vals[3])
        inv = 1.0 / tot
        off = (g // 8) * (N_REF * 128) + (g % 8) * L
        for j in range(N_REF):
            o_v[pl.ds(off + j * 128, L)] = s_vals[j] * inv
    pltpu.sync_copy(o_v, out_hbm.at[pl.ds(base * N_REF, TPW * N_REF)])


@functools.lru_cache(maxsize=1)
def _sc_rank():
    return pl.kernel(
        _sc_body,
        out_type=jax.ShapeDtypeStruct((B * N_REF,), jnp.float32),
        mesh=plsc.VectorSubcoreMesh(core_axis_name="c", subcore_axis_name="s",
                                    num_cores=NC, num_subcores=NS),
        compiler_params=pltpu.CompilerParams(needs_layout_passes=False,
                                             use_tc_tiling_on_sc=False),
        scratch_types=[
            pltpu.VMEM((8192,), jnp.float32),
            pltpu.VMEM((1 + N_REF, TPW), jnp.int32),
            pltpu.VMEM((TPW,), jnp.int32),
            pltpu.VMEM((TPW * N_REF,), jnp.float32),
            pltpu.SemaphoreType.DMA,
        ],
    )


def kernel(stimulus_set, kernel_gate_weights, table, w0, w1):
    sim = _sim_tables(table, w0, w1)
    sst = stimulus_set.T
    out_flat = _sc_rank()(sim.reshape(8192), sst, kernel_gate_weights)
    return (out_flat.reshape(B // 128, N_REF, 128)
            .transpose(0, 2, 1).reshape(B, N_REF))


# speedup vs baseline: 1.0039x; 1.0039x over previous
import functools

import jax
import jax.numpy as jnp
from jax import lax
from jax.experimental import pallas as pl
from jax.experimental.pallas import tpu as pltpu
from jax.experimental.pallas import tpu_sc as plsc

B = 16384
N_STIMULI = 30
N_DIM = 10
N_REF = 4
NV = N_STIMULI + 1

NC = 2
NS = 16
L = 16
NW = NC * NS
TPW = B // NW
CHUNKS = TPW // L


def _sim_body(table_ref, w0_ref, w1_ref, out_ref):
    t = table_ref[...]
    z1 = t[:, None, :]
    z2 = t[None, :, :]
    sq = (z1 - z2) * (z1 - z2)
    for g in range(2):
        w = (w0_ref if g == 0 else w1_ref)[...]
        d2 = jnp.sum(sq * w[None, None, :], axis=-1)
        s = jnp.exp(-jnp.sqrt(d2 + 1e-12))
        out_ref[g * 32:g * 32 + NV, :NV] = s


_sim_tables = pl.pallas_call(
    _sim_body,
    out_shape=jax.ShapeDtypeStruct((64, 128), jnp.float32),
)


def _sc_body(sim_hbm, sst_hbm, gate_hbm, out_hbm,
             sim_v, ss_v, gate_v, o_v, sem):
    cid = lax.axis_index("c")
    sid = lax.axis_index("s")
    wid = sid * NC + cid
    base = wid * TPW

    c1 = pltpu.async_copy(sim_hbm, sim_v, sem)
    c2 = pltpu.async_copy(sst_hbm.at[:, pl.ds(base, TPW)], ss_v, sem)
    c3 = pltpu.async_copy(gate_hbm.at[pl.ds(base, TPW)], gate_v, sem)
    c1.wait()
    c2.wait()
    c3.wait()

    @plsc.parallel_loop(0, CHUNKS, unroll=4)
    def chunk(g):
        sl = pl.ds(g * L, L)
        gq = gate_v[sl] * 4096 + ss_v[0, sl] * 128
        s_vals = [plsc.load_gather(sim_v, [gq + ss_v[1 + j, sl]])
                  for j in range(N_REF)]
        tot = (s_vals[0] + s_vals[1]) + (s_vals[2] + s_vals[3])
        inv = 1.0 / tot
        off = (g // 8) * (N_REF * 128) + (g % 8) * L
        for j in range(N_REF):
            o_v[pl.ds(off + j * 128, L)] = s_vals[j] * inv
    pltpu.sync_copy(o_v, out_hbm.at[pl.ds(base * N_REF, TPW * N_REF)])


@functools.lru_cache(maxsize=1)
def _sc_rank():
    return pl.kernel(
        _sc_body,
        out_type=jax.ShapeDtypeStruct((B * N_REF,), jnp.float32),
        mesh=plsc.VectorSubcoreMesh(core_axis_name="c", subcore_axis_name="s",
                                    num_cores=NC, num_subcores=NS),
        compiler_params=pltpu.CompilerParams(needs_layout_passes=False,
                                             use_tc_tiling_on_sc=False),
        scratch_types=[
            pltpu.VMEM((8192,), jnp.float32),
            pltpu.VMEM((1 + N_REF, TPW), jnp.int32),
            pltpu.VMEM((TPW,), jnp.int32),
            pltpu.VMEM((TPW * N_REF,), jnp.float32),
            pltpu.SemaphoreType.DMA,
        ],
    )


def kernel(stimulus_set, kernel_gate_weights, table, w0, w1):
    sim = _sim_tables(table, w0, w1)
    sst = stimulus_set.T
    out_flat = _sc_rank()(sim.reshape(8192), sst, kernel_gate_weights)
    return (out_flat.reshape(B // 128, N_REF, 128)
            .transpose(0, 2, 1).reshape(B, N_REF))
